# baseline (speedup 1.0000x reference)
import functools

import jax
import jax.numpy as jnp
from jax import lax
from jax.experimental import pallas as pl
from jax.experimental.pallas import tpu as pltpu
from jax.experimental.pallas import tpu_sc as plsc

N = 10000
E = 320000
D = 128
RH = 16
L = 4

NC = 2
NS = 16
NW = NC * NS
EPW = E // NW
CHUNK = 40
NCHUNK = EPW // CHUNK
NP = 10240
ROWS_PER_SUB = NP // NS

_HIGH = jax.lax.Precision.HIGHEST


def _r2_body(src_hbm, dst_hbm, px_hbm, py_hbm, pz_hbm, r2_hbm,
             srcb, dstb, pxb, pyb, pzb, r2b, sem):
    cid = lax.axis_index("c")
    sid = lax.axis_index("s")
    wid = sid * NC + cid
    base = wid * EPW
    pltpu.sync_copy(src_hbm.at[pl.ds(base, EPW)], srcb)
    pltpu.sync_copy(dst_hbm.at[pl.ds(base, EPW)], dstb)
    pltpu.sync_copy(px_hbm, pxb)
    pltpu.sync_copy(py_hbm, pyb)
    pltpu.sync_copy(pz_hbm, pzb)

    def grp(i, _):
        s16 = srcb[pl.ds(i * 16, 16)]
        d16 = dstb[pl.ds(i * 16, 16)]
        acc = jnp.zeros((16,), jnp.float32)
        for cb in (pxb, pyb, pzb):
            a = plsc.load_gather(cb, [s16])
            b = plsc.load_gather(cb, [d16])
            diff = b - a
            acc = acc + diff * diff
        r2b[pl.ds(i * 16, 16)] = acc
        return 0

    lax.fori_loop(0, EPW // 16, grp, 0)
    pltpu.sync_copy(r2b, r2_hbm.at[pl.ds(base, EPW)])


_r2_call = functools.partial(
    pl.kernel,
    out_type=jax.ShapeDtypeStruct((E,), jnp.float32),
    mesh=plsc.VectorSubcoreMesh(core_axis_name="c", subcore_axis_name="s"),
    scratch_types=[
        pltpu.VMEM((EPW,), jnp.int32),
        pltpu.VMEM((EPW,), jnp.int32),
        pltpu.VMEM((N,), jnp.float32),
        pltpu.VMEM((N,), jnp.float32),
        pltpu.VMEM((N,), jnp.float32),
        pltpu.VMEM((EPW,), jnp.float32),
        pltpu.SemaphoreType.DMA,
    ],
    compiler_params=pltpu.CompilerParams(needs_layout_passes=False),
)(_r2_body)


def _edge_body(hw_hbm, rad_hbm, src_hbm, dst_hbm, zeros_hbm, agg_hbm,
               srcb, dstb, gath, radb, acc,
               sem_g, sem_r, sem_s, sem_d, sem_i):
    cid = lax.axis_index("c")
    sid = lax.axis_index("s")
    wid = sid * NC + cid
    rowoff = sid * ROWS_PER_SUB
    pltpu.sync_copy(zeros_hbm.at[pl.ds(rowoff, ROWS_PER_SUB)],
                    acc.at[pl.ds(rowoff, ROWS_PER_SUB)])
    plsc.subcore_barrier()

    pltpu.sync_copy(src_hbm.at[wid, 0], srcb.at[0])
    pltpu.sync_copy(src_hbm.at[wid, 1], srcb.at[1])
    for u in range(2):
        pltpu.async_copy(hw_hbm.at[srcb.at[u]], gath.at[u], sem_g.at[u])
        pltpu.async_copy(rad_hbm.at[wid, u], radb.at[u], sem_r.at[u])
        pltpu.async_copy(dst_hbm.at[wid, u], dstb.at[u], sem_d.at[u])
    pltpu.async_copy(src_hbm.at[wid, 2], srcb.at[2], sem_i.at[2])

    def _wait(dst_ref, sem):
        pltpu.make_async_copy(hw_hbm.at[pl.ds(0, CHUNK)], dst_ref, sem).wait()

    def _wait_idx(dst_ref, sem):
        pltpu.make_async_copy(dst_hbm.at[wid, 0], dst_ref, sem).wait()

    def chunk(i, u, dyn):
        b = u % 3
        b2 = (u + 2) % 3
        bm1 = (u - 1) % 3
        _wait(gath.at[b], sem_g.at[b])
        _wait(radb.at[b], sem_r.at[b])

        @plsc.parallel_loop(0, CHUNK, 1, unroll=8)
        def _(j):
            for k in range(D // 16):
                sl = pl.ds(k * 16, 16)
                gath[b, j, sl] = gath[b, j, sl] * radb[b, j, sl]

        _wait_idx(dstb.at[b], sem_d.at[b])
        pltpu.async_copy(gath.at[b], acc.at[dstb.at[b]], sem_s.at[b],
                         add=True)

        def waits():
            _wait(gath.at[bm1], sem_s.at[bm1])

        if dyn and u == 0:
            pl.when(i > 0)(waits)
        else:
            waits()

        if dyn:
            @pl.when(i < NCHUNK - 2)
            def _():
                _wait_idx(srcb.at[b2], sem_i.at[b2])
                pltpu.async_copy(hw_hbm.at[srcb.at[b2]], gath.at[b2],
                                 sem_g.at[b2])
                pltpu.async_copy(rad_hbm.at[wid, i + 2], radb.at[b2],
                                 sem_r.at[b2])
                pltpu.async_copy(dst_hbm.at[wid, i + 2], dstb.at[b2],
                                 sem_d.at[b2])

            @pl.when(i < NCHUNK - 3)
            def _():
                pltpu.async_copy(src_hbm.at[wid, i + 3], srcb.at[b],
                                 sem_i.at[b])

    def triple(t, _):
        for u in range(3):
            chunk(t * 3 + u, u, True)
        return 0

    lax.fori_loop(0, NCHUNK // 3, triple, 0)
    for j in range(NCHUNK % 3):
        i = (NCHUNK // 3) * 3 + j
        chunk(i, i % 3, False)
    _wait(gath.at[(NCHUNK - 1) % 3], sem_s.at[(NCHUNK - 1) % 3])
    plsc.subcore_barrier()
    pltpu.sync_copy(acc.at[pl.ds(rowoff, ROWS_PER_SUB)],
                    agg_hbm.at[cid, pl.ds(rowoff, ROWS_PER_SUB)])


_edge_call = functools.partial(
    pl.kernel,
    out_type=jax.ShapeDtypeStruct((NC, NP, D), jnp.float32),
    mesh=plsc.VectorSubcoreMesh(core_axis_name="c", subcore_axis_name="s"),
    scratch_types=[
        pltpu.VMEM((3, CHUNK), jnp.int32),
        pltpu.VMEM((3, CHUNK), jnp.int32),
        pltpu.VMEM((3, CHUNK, D), jnp.float32),
        pltpu.VMEM((3, CHUNK, D), jnp.float32),
        pltpu.VMEM_SHARED((NP, D), jnp.float32),
        pltpu.SemaphoreType.DMA((3,)),
        pltpu.SemaphoreType.DMA((3,)),
        pltpu.SemaphoreType.DMA((3,)),
        pltpu.SemaphoreType.DMA((3,)),
        pltpu.SemaphoreType.DMA((3,)),
    ],
    compiler_params=pltpu.CompilerParams(needs_layout_passes=False),
)(_edge_body)


def _emb_body(an_ref, emb_ref, out_ref):
    an = an_ref[...]
    onehot = (an == lax.broadcasted_iota(jnp.int32, (an.shape[0], 128), 1))
    out_ref[...] = jnp.dot(onehot.astype(jnp.float32), emb_ref[...],
                           precision=_HIGH)


def _emb_lookup(an2, embp):
    B = 2000
    return pl.pallas_call(
        _emb_body,
        grid=(N // B,),
        in_specs=[
            pl.BlockSpec((B, 1), lambda i: (i, 0)),
            pl.BlockSpec((128, D), lambda i: (0, 0)),
        ],
        out_specs=pl.BlockSpec((B, D), lambda i: (i, 0)),
        out_shape=jax.ShapeDtypeStruct((N, D), jnp.float32),
    )(an2, embp)


def _rad_body(r2_ref, w1_ref, b1_ref, w2_ref, b2_ref, out_ref):
    r = jnp.sqrt(r2_ref[...] + 1e-8)
    a = jnp.maximum(r * w1_ref[...] + b1_ref[...], 0.0)
    out_ref[...] = jnp.dot(a, w2_ref[...]) + b2_ref[...]


def _rad_layer(r2c, w1, b1, w2, b2):
    BE = 8000
    return pl.pallas_call(
        _rad_body,
        grid=(E // BE,),
        in_specs=[
            pl.BlockSpec((BE, 1), lambda i: (i, 0)),
            pl.BlockSpec((1, RH), lambda i: (0, 0)),
            pl.BlockSpec((1, RH), lambda i: (0, 0)),
            pl.BlockSpec((RH, D), lambda i: (0, 0)),
            pl.BlockSpec((1, D), lambda i: (0, 0)),
        ],
        out_specs=pl.BlockSpec((BE, D), lambda i: (i, 0)),
        out_shape=jax.ShapeDtypeStruct((E, D), jnp.float32),
    )(r2c, w1, b1, w2, b2)


def _mm_body(h_ref, we_ref, ws_ref, hw_ref, hs_ref):
    h = h_ref[...]
    hw_ref[...] = jnp.dot(h, we_ref[...])
    hs_ref[...] = jnp.dot(h, ws_ref[...])


def _node_mm(h, we, ws):
    return pl.pallas_call(
        _mm_body,
        out_shape=(jax.ShapeDtypeStruct((N, D), jnp.float32),
                   jax.ShapeDtypeStruct((N, D), jnp.float32)),
    )(h, we, ws)


def _combine_body(agg_ref, hs_ref, ns_ref, out_ref, *, apply_norm):
    h = agg_ref[0, :N] + agg_ref[1, :N] + hs_ref[...]
    if apply_norm:
        nrm = jnp.sqrt(jnp.sum(h * h, axis=-1, keepdims=True) + 1e-12)
        h = (h / nrm) * jnp.maximum(ns_ref[...] * nrm, 0.0)
    out_ref[...] = h


def _combine(agg2, hself, ns_row, apply_norm):
    return pl.pallas_call(
        functools.partial(_combine_body, apply_norm=apply_norm),
        out_shape=jax.ShapeDtypeStruct((N, D), jnp.float32),
    )(agg2, hself, ns_row)


def _head_body(h_ref, w1_ref, b1_ref, w2t_ref, b2_ref, out_ref):
    pooled = jnp.max(h_ref[...], axis=0, keepdims=True)
    pooled8 = jnp.broadcast_to(pooled, (8, D))
    z = jnp.maximum(jnp.dot(pooled8, w1_ref[...])
                    + b1_ref[...], 0.0)
    val = jnp.sum(z[0:1, :] * w2t_ref[...], axis=-1, keepdims=True)
    out_ref[...] = val + b2_ref[...]


def _head(h, W1, b1r, W2t, b2r):
    return pl.pallas_call(
        _head_body,
        out_shape=jax.ShapeDtypeStruct((1, 1), jnp.float32),
    )(h, W1, b1r, W2t, b2r)


@jax.jit
def kernel(pos, edge_index, atomic_numbers, emb, Wedge, Wself, Rw1, Rb1,
           Rw2, Rb2, norm_scale, W1, b1, W2, b2):
    src = edge_index[0].astype(jnp.int32)
    dst = edge_index[1].astype(jnp.int32)
    src3 = src.reshape(NW, NCHUNK, CHUNK)
    dst3 = dst.reshape(NW, NCHUNK, CHUNK)
    an2 = atomic_numbers.astype(jnp.int32).reshape(N, 1)
    embp = jnp.zeros((128, D), jnp.float32).at[:100].set(emb)
    zeros_nd = jnp.zeros((NP, D), jnp.float32)

    h = _emb_lookup(an2, embp)
    posx = jnp.asarray(pos[:, 0])
    posy = jnp.asarray(pos[:, 1])
    posz = jnp.asarray(pos[:, 2])
    r2 = _r2_call(src, dst, posx, posy, posz)
    r2c = r2.reshape(E, 1)

    rad4s = [
        _rad_layer(r2c, Rw1[l], Rb1[l].reshape(1, RH), Rw2[l],
                   Rb2[l].reshape(1, D)).reshape(NW, NCHUNK, CHUNK, D)
        for l in range(L)
    ]

    for l in range(L):
        rad4 = rad4s[l]
        hw, hself = _node_mm(h, Wedge[l], Wself[l])
        agg2 = _edge_call(hw, rad4, src3, dst3, zeros_nd)
        h = _combine(agg2, hself, norm_scale[l].reshape(1, D) if l < L - 1
                     else jnp.zeros((1, D), jnp.float32), l < L - 1)

    out = _head(h, W1, b1.reshape(1, D), W2.reshape(1, D), b2.reshape(1, 1))
    return out.reshape(1)


# speedup vs baseline: 1.0203x; 1.0203x over previous
import functools

import jax
import jax.numpy as jnp
from jax import lax
from jax.experimental import pallas as pl
from jax.experimental.pallas import tpu as pltpu
from jax.experimental.pallas import tpu_sc as plsc

N = 10000
E = 320000
D = 128
RH = 16
L = 4

NC = 2
NS = 16
NW = NC * NS
EPW = E // NW
CHUNK = 80
NCHUNK = EPW // CHUNK
NP = 10240
ROWS_PER_SUB = NP // NS

_HIGH = jax.lax.Precision.HIGHEST


def _r2_body(src_hbm, dst_hbm, px_hbm, py_hbm, pz_hbm, r2_hbm,
             srcb, dstb, pxb, pyb, pzb, r2b, sem):
    cid = lax.axis_index("c")
    sid = lax.axis_index("s")
    wid = sid * NC + cid
    base = wid * EPW
    pltpu.sync_copy(src_hbm.at[pl.ds(base, EPW)], srcb)
    pltpu.sync_copy(dst_hbm.at[pl.ds(base, EPW)], dstb)
    pltpu.sync_copy(px_hbm, pxb)
    pltpu.sync_copy(py_hbm, pyb)
    pltpu.sync_copy(pz_hbm, pzb)

    def grp(i, _):
        s16 = srcb[pl.ds(i * 16, 16)]
        d16 = dstb[pl.ds(i * 16, 16)]
        acc = jnp.zeros((16,), jnp.float32)
        for cb in (pxb, pyb, pzb):
            a = plsc.load_gather(cb, [s16])
            b = plsc.load_gather(cb, [d16])
            diff = b - a
            acc = acc + diff * diff
        r2b[pl.ds(i * 16, 16)] = acc
        return 0

    lax.fori_loop(0, EPW // 16, grp, 0)
    pltpu.sync_copy(r2b, r2_hbm.at[pl.ds(base, EPW)])


_r2_call = functools.partial(
    pl.kernel,
    out_type=jax.ShapeDtypeStruct((E,), jnp.float32),
    mesh=plsc.VectorSubcoreMesh(core_axis_name="c", subcore_axis_name="s"),
    scratch_types=[
        pltpu.VMEM((EPW,), jnp.int32),
        pltpu.VMEM((EPW,), jnp.int32),
        pltpu.VMEM((N,), jnp.float32),
        pltpu.VMEM((N,), jnp.float32),
        pltpu.VMEM((N,), jnp.float32),
        pltpu.VMEM((EPW,), jnp.float32),
        pltpu.SemaphoreType.DMA,
    ],
    compiler_params=pltpu.CompilerParams(needs_layout_passes=False),
)(_r2_body)


def _edge_body(hw_hbm, rad_hbm, src_hbm, dst_hbm, agg_hbm,
               srcb, dstb, gath, radb, acc,
               sem_g, sem_r, sem_s, sem_d, sem_i):
    cid = lax.axis_index("c")
    sid = lax.axis_index("s")
    wid = sid * NC + cid
    rowoff = sid * ROWS_PER_SUB

    zv = jnp.zeros((16,), jnp.float32)

    @plsc.parallel_loop(0, CHUNK, 1, unroll=8)
    def _(j):
        for k in range(D // 16):
            gath[0, j, pl.ds(k * 16, 16)] = zv

    for z in range(ROWS_PER_SUB // CHUNK):
        pltpu.sync_copy(gath.at[0],
                        acc.at[pl.ds(rowoff + z * CHUNK, CHUNK)])
    plsc.subcore_barrier()

    pltpu.sync_copy(src_hbm.at[wid, 0], srcb.at[0])
    pltpu.async_copy(hw_hbm.at[srcb.at[0]], gath.at[0], sem_g)
    pltpu.async_copy(rad_hbm.at[wid, 0], radb.at[0], sem_r)
    pltpu.async_copy(dst_hbm.at[wid, 0], dstb.at[0], sem_d)
    pltpu.async_copy(src_hbm.at[wid, 1], srcb.at[1], sem_i)

    def _wait(dst_ref, sem):
        pltpu.make_async_copy(hw_hbm.at[pl.ds(0, CHUNK)], dst_ref, sem).wait()

    def _wait_idx(dst_ref, sem):
        pltpu.make_async_copy(dst_hbm.at[wid, 0], dst_ref, sem).wait()

    def chunk(i, p, in_loop):
        _wait(gath.at[p], sem_g)
        _wait(radb.at[p], sem_r)

        if in_loop:
            @pl.when(i > 0)
            def _():
                _wait(gath.at[1 - p], sem_s)

            _wait_idx(srcb.at[1 - p], sem_i)
            pltpu.async_copy(hw_hbm.at[srcb.at[1 - p]], gath.at[1 - p],
                             sem_g)
            pltpu.async_copy(rad_hbm.at[wid, i + 1], radb.at[1 - p], sem_r)
            pltpu.async_copy(dst_hbm.at[wid, i + 1], dstb.at[1 - p], sem_d)

            @pl.when(i < NCHUNK - 2)
            def _():
                pltpu.async_copy(src_hbm.at[wid, i + 2], srcb.at[p], sem_i)
        else:
            _wait(gath.at[1 - p], sem_s)

        @plsc.parallel_loop(0, CHUNK, 1, unroll=8)
        def _(j):
            for k in range(D // 16):
                sl = pl.ds(k * 16, 16)
                gath[p, j, sl] = gath[p, j, sl] * radb[p, j, sl]

        _wait_idx(dstb.at[p], sem_d)
        pltpu.async_copy(gath.at[p], acc.at[dstb.at[p]], sem_s, add=True)

    def pair(i2, _):
        chunk(i2 * 2, 0, True)
        chunk(i2 * 2 + 1, 1, True)
        return 0

    lax.fori_loop(0, (NCHUNK - 1) // 2, pair, 0)
    chunk(NCHUNK - 1, (NCHUNK - 1) % 2, False)
    _wait(gath.at[(NCHUNK - 1) % 2], sem_s)
    plsc.subcore_barrier()
    pltpu.sync_copy(acc.at[pl.ds(rowoff, ROWS_PER_SUB)],
                    agg_hbm.at[cid, pl.ds(rowoff, ROWS_PER_SUB)])


_edge_call = functools.partial(
    pl.kernel,
    out_type=jax.ShapeDtypeStruct((NC, NP, D), jnp.float32),
    mesh=plsc.VectorSubcoreMesh(core_axis_name="c", subcore_axis_name="s"),
    scratch_types=[
        pltpu.VMEM((2, CHUNK), jnp.int32),
        pltpu.VMEM((2, CHUNK), jnp.int32),
        pltpu.VMEM((2, CHUNK, D), jnp.float32),
        pltpu.VMEM((2, CHUNK, D), jnp.float32),
        pltpu.VMEM_SHARED((NP, D), jnp.float32),
        pltpu.SemaphoreType.DMA,
        pltpu.SemaphoreType.DMA,
        pltpu.SemaphoreType.DMA,
        pltpu.SemaphoreType.DMA,
        pltpu.SemaphoreType.DMA,
    ],
    compiler_params=pltpu.CompilerParams(needs_layout_passes=False),
)(_edge_body)


def _emb_body(an_ref, emb_ref, out_ref):
    an = an_ref[...]
    onehot = (an == lax.broadcasted_iota(jnp.int32, (an.shape[0], 128), 1))
    out_ref[...] = jnp.dot(onehot.astype(jnp.float32), emb_ref[...],
                           precision=_HIGH)


def _emb_lookup(an2, embp):
    B = 2000
    return pl.pallas_call(
        _emb_body,
        grid=(N // B,),
        in_specs=[
            pl.BlockSpec((B, 1), lambda i: (i, 0)),
            pl.BlockSpec((128, D), lambda i: (0, 0)),
        ],
        out_specs=pl.BlockSpec((B, D), lambda i: (i, 0)),
        out_shape=jax.ShapeDtypeStruct((N, D), jnp.float32),
    )(an2, embp)


def _rad_body(r2_ref, w1_ref, b1_ref, w2_ref, b2_ref, out_ref):
    r = jnp.sqrt(r2_ref[...] + 1e-8)
    a = jnp.maximum(r * w1_ref[...] + b1_ref[...], 0.0)
    out_ref[...] = jnp.dot(a, w2_ref[...]) + b2_ref[...]


def _rad_layer(r2c, w1, b1, w2, b2):
    BE = 8000
    return pl.pallas_call(
        _rad_body,
        grid=(E // BE,),
        in_specs=[
            pl.BlockSpec((BE, 1), lambda i: (i, 0)),
            pl.BlockSpec((1, RH), lambda i: (0, 0)),
            pl.BlockSpec((1, RH), lambda i: (0, 0)),
            pl.BlockSpec((RH, D), lambda i: (0, 0)),
            pl.BlockSpec((1, D), lambda i: (0, 0)),
        ],
        out_specs=pl.BlockSpec((BE, D), lambda i: (i, 0)),
        out_shape=jax.ShapeDtypeStruct((E, D), jnp.float32),
    )(r2c, w1, b1, w2, b2)


def _mm_body(h_ref, we_ref, ws_ref, hw_ref, hs_ref):
    h = h_ref[...]
    hw_ref[...] = jnp.dot(h, we_ref[...])
    hs_ref[...] = jnp.dot(h, ws_ref[...])


def _node_mm(h, we, ws):
    return pl.pallas_call(
        _mm_body,
        out_shape=(jax.ShapeDtypeStruct((N, D), jnp.float32),
                   jax.ShapeDtypeStruct((N, D), jnp.float32)),
    )(h, we, ws)


def _combine_body(agg_ref, hs_ref, ns_ref, out_ref, *, apply_norm):
    h = agg_ref[0, :N] + agg_ref[1, :N] + hs_ref[...]
    if apply_norm:
        nrm = jnp.sqrt(jnp.sum(h * h, axis=-1, keepdims=True) + 1e-12)
        h = (h / nrm) * jnp.maximum(ns_ref[...] * nrm, 0.0)
    out_ref[...] = h


def _combine(agg2, hself, ns_row, apply_norm):
    return pl.pallas_call(
        functools.partial(_combine_body, apply_norm=apply_norm),
        out_shape=jax.ShapeDtypeStruct((N, D), jnp.float32),
    )(agg2, hself, ns_row)


def _comb_mm_body(agg_ref, hs_ref, ns_ref, we_ref, ws_ref,
                  h_ref, hw_ref, hs2_ref):
    h = agg_ref[0, :N] + agg_ref[1, :N] + hs_ref[...]
    nrm = jnp.sqrt(jnp.sum(h * h, axis=-1, keepdims=True) + 1e-12)
    h = (h / nrm) * jnp.maximum(ns_ref[...] * nrm, 0.0)
    h_ref[...] = h
    hw_ref[...] = jnp.dot(h, we_ref[...])
    hs2_ref[...] = jnp.dot(h, ws_ref[...])


def _comb_mm(agg2, hself, ns_row, we, ws):
    return pl.pallas_call(
        _comb_mm_body,
        out_shape=(jax.ShapeDtypeStruct((N, D), jnp.float32),
                   jax.ShapeDtypeStruct((N, D), jnp.float32),
                   jax.ShapeDtypeStruct((N, D), jnp.float32)),
    )(agg2, hself, ns_row, we, ws)


def _head_body(h_ref, w1_ref, b1_ref, w2t_ref, b2_ref, out_ref):
    pooled = jnp.max(h_ref[...], axis=0, keepdims=True)
    pooled8 = jnp.broadcast_to(pooled, (8, D))
    z = jnp.maximum(jnp.dot(pooled8, w1_ref[...])
                    + b1_ref[...], 0.0)
    val = jnp.sum(z[0:1, :] * w2t_ref[...], axis=-1, keepdims=True)
    out_ref[...] = val + b2_ref[...]


def _head(h, W1, b1r, W2t, b2r):
    return pl.pallas_call(
        _head_body,
        out_shape=jax.ShapeDtypeStruct((1, 1), jnp.float32),
    )(h, W1, b1r, W2t, b2r)


@jax.jit
def kernel(pos, edge_index, atomic_numbers, emb, Wedge, Wself, Rw1, Rb1,
           Rw2, Rb2, norm_scale, W1, b1, W2, b2):
    src = edge_index[0].astype(jnp.int32)
    dst = edge_index[1].astype(jnp.int32)
    src3 = src.reshape(NW, NCHUNK, CHUNK)
    dst3 = dst.reshape(NW, NCHUNK, CHUNK)
    an2 = atomic_numbers.astype(jnp.int32).reshape(N, 1)
    embp = jnp.zeros((128, D), jnp.float32).at[:100].set(emb)

    h = _emb_lookup(an2, embp)
    posx = jnp.asarray(pos[:, 0])
    posy = jnp.asarray(pos[:, 1])
    posz = jnp.asarray(pos[:, 2])
    r2 = _r2_call(src, dst, posx, posy, posz)
    r2c = r2.reshape(E, 1)

    rad4s = [
        _rad_layer(r2c, Rw1[l], Rb1[l].reshape(1, RH), Rw2[l],
                   Rb2[l].reshape(1, D)).reshape(NW, NCHUNK, CHUNK, D)
        for l in range(L)
    ]

    hw, hself = _node_mm(h, Wedge[0], Wself[0])
    for l in range(L):
        agg2 = _edge_call(hw, rad4s[l], src3, dst3)
        if l < L - 1:
            h, hw, hself = _comb_mm(agg2, hself,
                                    norm_scale[l].reshape(1, D),
                                    Wedge[l + 1], Wself[l + 1])
        else:
            h = _combine(agg2, hself, jnp.zeros((1, D), jnp.float32), False)

    out = _head(h, W1, b1.reshape(1, D), W2.reshape(1, D), b2.reshape(1, 1))
    return out.reshape(1)


# speedup vs baseline: 1.0280x; 1.0076x over previous
import functools

import jax
import jax.numpy as jnp
from jax import lax
from jax.experimental import pallas as pl
from jax.experimental.pallas import tpu as pltpu
from jax.experimental.pallas import tpu_sc as plsc

N = 10000
E = 320000
D = 128
RH = 16
L = 4

NC = 2
NS = 16
NW = NC * NS
EPW = E // NW
CHUNK = 80
NCHUNK = EPW // CHUNK
NP = 10240
ROWS_PER_SUB = NP // NS

_HIGH = jax.lax.Precision.HIGHEST


def _r2_body(src_hbm, dst_hbm, px_hbm, py_hbm, pz_hbm, r2_hbm,
             srcb, dstb, pxb, pyb, pzb, r2b, sem):
    cid = lax.axis_index("c")
    sid = lax.axis_index("s")
    wid = sid * NC + cid
    base = wid * EPW
    pltpu.sync_copy(src_hbm.at[pl.ds(base, EPW)], srcb)
    pltpu.sync_copy(dst_hbm.at[pl.ds(base, EPW)], dstb)
    pltpu.sync_copy(px_hbm, pxb)
    pltpu.sync_copy(py_hbm, pyb)
    pltpu.sync_copy(pz_hbm, pzb)

    def grp(i, _):
        s16 = srcb[pl.ds(i * 16, 16)]
        d16 = dstb[pl.ds(i * 16, 16)]
        acc = jnp.zeros((16,), jnp.float32)
        for cb in (pxb, pyb, pzb):
            a = plsc.load_gather(cb, [s16])
            b = plsc.load_gather(cb, [d16])
            diff = b - a
            acc = acc + diff * diff
        r2b[pl.ds(i * 16, 16)] = acc
        return 0

    lax.fori_loop(0, EPW // 16, grp, 0)
    pltpu.sync_copy(r2b, r2_hbm.at[pl.ds(base, EPW)])


_r2_call = functools.partial(
    pl.kernel,
    out_type=jax.ShapeDtypeStruct((E,), jnp.float32),
    mesh=plsc.VectorSubcoreMesh(core_axis_name="c", subcore_axis_name="s"),
    scratch_types=[
        pltpu.VMEM((EPW,), jnp.int32),
        pltpu.VMEM((EPW,), jnp.int32),
        pltpu.VMEM((N,), jnp.float32),
        pltpu.VMEM((N,), jnp.float32),
        pltpu.VMEM((N,), jnp.float32),
        pltpu.VMEM((EPW,), jnp.float32),
        pltpu.SemaphoreType.DMA,
    ],
    compiler_params=pltpu.CompilerParams(needs_layout_passes=False),
)(_r2_body)


def _edge_body(hw_hbm, rad_hbm, src_hbm, dst_hbm, agg_hbm,
               srcb, dstb, gath, radb, acc,
               sem_g, sem_r, sem_s, sem_d, sem_i):
    cid = lax.axis_index("c")
    sid = lax.axis_index("s")
    wid = sid * NC + cid
    rowoff = sid * ROWS_PER_SUB

    zv = jnp.zeros((16,), jnp.float32)

    @plsc.parallel_loop(0, CHUNK, 1, unroll=8)
    def _(j):
        for k in range(D // 16):
            gath[0, j, pl.ds(k * 16, 16)] = zv

    for z in range(ROWS_PER_SUB // CHUNK):
        pltpu.sync_copy(gath.at[0],
                        acc.at[pl.ds(rowoff + z * CHUNK, CHUNK)])
    plsc.subcore_barrier()

    pltpu.sync_copy(src_hbm.at[wid, 0], srcb.at[0])
    pltpu.async_copy(hw_hbm.at[srcb.at[0]], gath.at[0], sem_g)
    pltpu.async_copy(rad_hbm.at[wid, 0], radb.at[0], sem_r)
    pltpu.async_copy(dst_hbm.at[wid, 0], dstb.at[0], sem_d)
    pltpu.async_copy(src_hbm.at[wid, 1], srcb.at[1], sem_i)

    def _wait(dst_ref, sem):
        pltpu.make_async_copy(hw_hbm.at[pl.ds(0, CHUNK)], dst_ref, sem).wait()

    def _wait_idx(dst_ref, sem):
        pltpu.make_async_copy(dst_hbm.at[wid, 0], dst_ref, sem).wait()

    def chunk(i, p, in_loop):
        _wait(gath.at[p], sem_g)
        _wait(radb.at[p], sem_r)

        if in_loop:
            @pl.when(i > 0)
            def _():
                _wait(gath.at[1 - p], sem_s)

            _wait_idx(srcb.at[1 - p], sem_i)
            pltpu.async_copy(hw_hbm.at[srcb.at[1 - p]], gath.at[1 - p],
                             sem_g)
            pltpu.async_copy(rad_hbm.at[wid, i + 1], radb.at[1 - p], sem_r)
            pltpu.async_copy(dst_hbm.at[wid, i + 1], dstb.at[1 - p], sem_d)

            @pl.when(i < NCHUNK - 2)
            def _():
                pltpu.async_copy(src_hbm.at[wid, i + 2], srcb.at[p], sem_i)
        else:
            _wait(gath.at[1 - p], sem_s)

        @plsc.parallel_loop(0, CHUNK, 1, unroll=8)
        def _(j):
            for k in range(D // 16):
                sl = pl.ds(k * 16, 16)
                gath[p, j, sl] = gath[p, j, sl] * radb[p, j, sl]

        _wait_idx(dstb.at[p], sem_d)
        pltpu.async_copy(gath.at[p], acc.at[dstb.at[p]], sem_s, add=True)

    def pair(i2, _):
        chunk(i2 * 2, 0, True)
        chunk(i2 * 2 + 1, 1, True)
        return 0

    lax.fori_loop(0, (NCHUNK - 1) // 2, pair, 0)
    chunk(NCHUNK - 1, (NCHUNK - 1) % 2, False)
    _wait(gath.at[(NCHUNK - 1) % 2], sem_s)
    plsc.subcore_barrier()
    pltpu.sync_copy(acc.at[pl.ds(rowoff, ROWS_PER_SUB)],
                    agg_hbm.at[cid, pl.ds(rowoff, ROWS_PER_SUB)])


_edge_call = functools.partial(
    pl.kernel,
    out_type=jax.ShapeDtypeStruct((NC, NP, D), jnp.float32),
    mesh=plsc.VectorSubcoreMesh(core_axis_name="c", subcore_axis_name="s"),
    scratch_types=[
        pltpu.VMEM((2, CHUNK), jnp.int32),
        pltpu.VMEM((2, CHUNK), jnp.int32),
        pltpu.VMEM((2, CHUNK, D), jnp.float32),
        pltpu.VMEM((2, CHUNK, D), jnp.float32),
        pltpu.VMEM_SHARED((NP, D), jnp.float32),
        pltpu.SemaphoreType.DMA,
        pltpu.SemaphoreType.DMA,
        pltpu.SemaphoreType.DMA,
        pltpu.SemaphoreType.DMA,
        pltpu.SemaphoreType.DMA,
    ],
    compiler_params=pltpu.CompilerParams(needs_layout_passes=False),
)(_edge_body)


def _emb_mm_body(an_ref, emb_ref, we_ref, ws_ref, h_ref, hw_ref, hs_ref):
    an = an_ref[...]
    onehot = (an == lax.broadcasted_iota(jnp.int32, (an.shape[0], 128), 1))
    h = jnp.dot(onehot.astype(jnp.float32), emb_ref[...], precision=_HIGH)
    h_ref[...] = h
    hw_ref[...] = jnp.dot(h, we_ref[...])
    hs_ref[...] = jnp.dot(h, ws_ref[...])


def _emb_mm(an2, embp, we, ws):
    return pl.pallas_call(
        _emb_mm_body,
        out_shape=(jax.ShapeDtypeStruct((N, D), jnp.float32),
                   jax.ShapeDtypeStruct((N, D), jnp.float32),
                   jax.ShapeDtypeStruct((N, D), jnp.float32)),
    )(an2, embp, we, ws)


def _rad_body(r2_ref, w1_ref, b1_ref, w2_ref, b2_ref, out_ref):
    r = jnp.sqrt(r2_ref[...] + 1e-8)
    a = jnp.maximum(r * w1_ref[...] + b1_ref[...], 0.0)
    out_ref[...] = jnp.dot(a, w2_ref[...]) + b2_ref[...]


def _rad_layer(r2c, w1, b1, w2, b2):
    BE = 8000
    return pl.pallas_call(
        _rad_body,
        grid=(E // BE,),
        in_specs=[
            pl.BlockSpec((BE, 1), lambda i: (i, 0)),
            pl.BlockSpec((1, RH), lambda i: (0, 0)),
            pl.BlockSpec((1, RH), lambda i: (0, 0)),
            pl.BlockSpec((RH, D), lambda i: (0, 0)),
            pl.BlockSpec((1, D), lambda i: (0, 0)),
        ],
        out_specs=pl.BlockSpec((BE, D), lambda i: (i, 0)),
        out_shape=jax.ShapeDtypeStruct((E, D), jnp.float32),
    )(r2c, w1, b1, w2, b2)


def _mm_body(h_ref, we_ref, ws_ref, hw_ref, hs_ref):
    h = h_ref[...]
    hw_ref[...] = jnp.dot(h, we_ref[...])
    hs_ref[...] = jnp.dot(h, ws_ref[...])


def _node_mm(h, we, ws):
    return pl.pallas_call(
        _mm_body,
        out_shape=(jax.ShapeDtypeStruct((N, D), jnp.float32),
                   jax.ShapeDtypeStruct((N, D), jnp.float32)),
    )(h, we, ws)


def _comb_head_body(agg_ref, hs_ref, w1_ref, b1_ref, w2t_ref, b2_ref,
                    out_ref):
    h = agg_ref[0, :N] + agg_ref[1, :N] + hs_ref[...]
    pooled = jnp.max(h, axis=0, keepdims=True)
    pooled8 = jnp.broadcast_to(pooled, (8, D))
    z = jnp.maximum(jnp.dot(pooled8, w1_ref[...]) + b1_ref[...], 0.0)
    val = jnp.sum(z[0:1, :] * w2t_ref[...], axis=-1, keepdims=True)
    out_ref[...] = val + b2_ref[...]


def _comb_head(agg2, hself, W1, b1r, W2t, b2r):
    return pl.pallas_call(
        _comb_head_body,
        out_shape=jax.ShapeDtypeStruct((1, 1), jnp.float32),
    )(agg2, hself, W1, b1r, W2t, b2r)


def _comb_mm_body(agg_ref, hs_ref, ns_ref, we_ref, ws_ref,
                  h_ref, hw_ref, hs2_ref):
    h = agg_ref[0, :N] + agg_ref[1, :N] + hs_ref[...]
    nrm = jnp.sqrt(jnp.sum(h * h, axis=-1, keepdims=True) + 1e-12)
    h = (h / nrm) * jnp.maximum(ns_ref[...] * nrm, 0.0)
    h_ref[...] = h
    hw_ref[...] = jnp.dot(h, we_ref[...])
    hs2_ref[...] = jnp.dot(h, ws_ref[...])


def _comb_mm(agg2, hself, ns_row, we, ws):
    return pl.pallas_call(
        _comb_mm_body,
        out_shape=(jax.ShapeDtypeStruct((N, D), jnp.float32),
                   jax.ShapeDtypeStruct((N, D), jnp.float32),
                   jax.ShapeDtypeStruct((N, D), jnp.float32)),
    )(agg2, hself, ns_row, we, ws)


@jax.jit
def kernel(pos, edge_index, atomic_numbers, emb, Wedge, Wself, Rw1, Rb1,
           Rw2, Rb2, norm_scale, W1, b1, W2, b2):
    src = edge_index[0].astype(jnp.int32)
    dst = edge_index[1].astype(jnp.int32)
    src3 = src.reshape(NW, NCHUNK, CHUNK)
    dst3 = dst.reshape(NW, NCHUNK, CHUNK)
    an2 = atomic_numbers.astype(jnp.int32).reshape(N, 1)
    embp = jnp.zeros((128, D), jnp.float32).at[:100].set(emb)

    h, hw, hself = _emb_mm(an2, embp, Wedge[0], Wself[0])
    posx = jnp.asarray(pos[:, 0])
    posy = jnp.asarray(pos[:, 1])
    posz = jnp.asarray(pos[:, 2])
    r2 = _r2_call(src, dst, posx, posy, posz)
    r2c = r2.reshape(E, 1)

    rad4s = [
        _rad_layer(r2c, Rw1[l], Rb1[l].reshape(1, RH), Rw2[l],
                   Rb2[l].reshape(1, D)).reshape(NW, NCHUNK, CHUNK, D)
        for l in range(L)
    ]

    for l in range(L):
        agg2 = _edge_call(hw, rad4s[l], src3, dst3)
        if l < L - 1:
            h, hw, hself = _comb_mm(agg2, hself,
                                    norm_scale[l].reshape(1, D),
                                    Wedge[l + 1], Wself[l + 1])
        else:
            out = _comb_head(agg2, hself, W1, b1.reshape(1, D),
                             W2.reshape(1, D), b2.reshape(1, 1))
    return out.reshape(1)


# speedup vs baseline: 1.0385x; 1.0101x over previous
import functools

import jax
import jax.numpy as jnp
from jax import lax
from jax.experimental import pallas as pl
from jax.experimental.pallas import tpu as pltpu
from jax.experimental.pallas import tpu_sc as plsc

N = 10000
E = 320000
D = 128
RH = 16
L = 4

NC = 2
NS = 16
NW = NC * NS
EPW = E // NW
CHUNK = 40
NCHUNK = EPW // CHUNK
NP = 10240
ROWS_PER_SUB = NP // NS

_HIGH = jax.lax.Precision.HIGHEST


def _r2_body(src_hbm, dst_hbm, px_hbm, py_hbm, pz_hbm, r2_hbm,
             srcb, dstb, pxb, pyb, pzb, r2b, sem):
    cid = lax.axis_index("c")
    sid = lax.axis_index("s")
    wid = sid * NC + cid
    base = wid * EPW
    pltpu.sync_copy(src_hbm.at[pl.ds(base, EPW)], srcb)
    pltpu.sync_copy(dst_hbm.at[pl.ds(base, EPW)], dstb)
    pltpu.sync_copy(px_hbm, pxb)
    pltpu.sync_copy(py_hbm, pyb)
    pltpu.sync_copy(pz_hbm, pzb)

    def grp(i, _):
        s16 = srcb[pl.ds(i * 16, 16)]
        d16 = dstb[pl.ds(i * 16, 16)]
        acc = jnp.zeros((16,), jnp.float32)
        for cb in (pxb, pyb, pzb):
            a = plsc.load_gather(cb, [s16])
            b = plsc.load_gather(cb, [d16])
            diff = b - a
            acc = acc + diff * diff
        r2b[pl.ds(i * 16, 16)] = acc
        return 0

    lax.fori_loop(0, EPW // 16, grp, 0)
    pltpu.sync_copy(r2b, r2_hbm.at[pl.ds(base, EPW)])


_r2_call = functools.partial(
    pl.kernel,
    out_type=jax.ShapeDtypeStruct((E,), jnp.float32),
    mesh=plsc.VectorSubcoreMesh(core_axis_name="c", subcore_axis_name="s"),
    scratch_types=[
        pltpu.VMEM((EPW,), jnp.int32),
        pltpu.VMEM((EPW,), jnp.int32),
        pltpu.VMEM((N,), jnp.float32),
        pltpu.VMEM((N,), jnp.float32),
        pltpu.VMEM((N,), jnp.float32),
        pltpu.VMEM((EPW,), jnp.float32),
        pltpu.SemaphoreType.DMA,
    ],
    compiler_params=pltpu.CompilerParams(needs_layout_passes=False),
)(_r2_body)


def _edge_body(hw_hbm, rad_hbm, src_hbm, dst_hbm, agg_hbm,
               srcb, dstb, gath, radb, msgb, acc,
               sem_g, sem_r, sem_s, sem_d, sem_i):
    cid = lax.axis_index("c")
    sid = lax.axis_index("s")
    wid = sid * NC + cid
    rowoff = sid * ROWS_PER_SUB

    zv = jnp.zeros((16,), jnp.float32)

    @plsc.parallel_loop(0, CHUNK, 1, unroll=8)
    def _(j):
        for k in range(D // 16):
            gath[0, j, pl.ds(k * 16, 16)] = zv

    for z in range(ROWS_PER_SUB // CHUNK):
        pltpu.sync_copy(gath.at[0],
                        acc.at[pl.ds(rowoff + z * CHUNK, CHUNK)])
    plsc.subcore_barrier()

    pltpu.sync_copy(src_hbm.at[wid, 0], srcb.at[0])
    pltpu.async_copy(hw_hbm.at[srcb.at[0]], gath.at[0], sem_g.at[0])
    pltpu.async_copy(rad_hbm.at[wid, 0], radb.at[0], sem_r.at[0])
    pltpu.async_copy(dst_hbm.at[wid, 0], dstb.at[0], sem_d.at[0])
    pltpu.async_copy(src_hbm.at[wid, 1], srcb.at[1], sem_i)

    def _wait(dst_ref, sem):
        pltpu.make_async_copy(hw_hbm.at[pl.ds(0, CHUNK)], dst_ref, sem).wait()

    def _wait_idx(dst_ref, sem):
        pltpu.make_async_copy(dst_hbm.at[wid, 0], dst_ref, sem).wait()

    def chunk(i, u, pre, srcpre, dyn):
        p = u % 2
        d4 = u % 4
        if pre:
            _wait_idx(srcb.at[1 - p], sem_i)
            pltpu.async_copy(hw_hbm.at[srcb.at[1 - p]], gath.at[1 - p],
                             sem_g.at[1 - p])
            pltpu.async_copy(rad_hbm.at[wid, i + 1], radb.at[1 - p],
                             sem_r.at[1 - p])
            pltpu.async_copy(dst_hbm.at[wid, i + 1], dstb.at[(u + 1) % 4],
                             sem_d.at[(u + 1) % 4])
        if srcpre:
            pltpu.async_copy(src_hbm.at[wid, i + 2], srcb.at[p], sem_i)

        _wait(gath.at[p], sem_g.at[p])
        _wait(radb.at[p], sem_r.at[p])

        def waits():
            _wait(msgb.at[p], sem_s.at[p])

        if dyn and u < 2:
            pl.when(i >= 2)(waits)
        else:
            waits()

        @plsc.parallel_loop(0, CHUNK, 1, unroll=8)
        def _(j):
            for k in range(D // 16):
                sl = pl.ds(k * 16, 16)
                msgb[p, j, sl] = gath[p, j, sl] * radb[p, j, sl]

        _wait_idx(dstb.at[d4], sem_d.at[d4])
        pltpu.async_copy(msgb.at[p], acc.at[dstb.at[d4]], sem_s.at[p],
                         add=True)

    def quad(t, _):
        for u in range(4):
            chunk(t * 4 + u, u, True, True, True)
        return 0

    lax.fori_loop(0, (NCHUNK - 2) // 4, quad, 0)
    chunk(NCHUNK - 2, 0, True, False, False)
    chunk(NCHUNK - 1, 1, False, False, False)
    _wait(msgb.at[0], sem_s.at[0])
    _wait(msgb.at[1], sem_s.at[1])
    plsc.subcore_barrier()
    pltpu.sync_copy(acc.at[pl.ds(rowoff, ROWS_PER_SUB)],
                    agg_hbm.at[cid, pl.ds(rowoff, ROWS_PER_SUB)])


_edge_call = functools.partial(
    pl.kernel,
    out_type=jax.ShapeDtypeStruct((NC, NP, D), jnp.float32),
    mesh=plsc.VectorSubcoreMesh(core_axis_name="c", subcore_axis_name="s"),
    scratch_types=[
        pltpu.VMEM((2, CHUNK), jnp.int32),
        pltpu.VMEM((4, CHUNK), jnp.int32),
        pltpu.VMEM((2, CHUNK, D), jnp.float32),
        pltpu.VMEM((2, CHUNK, D), jnp.float32),
        pltpu.VMEM((2, CHUNK, D), jnp.float32),
        pltpu.VMEM_SHARED((NP, D), jnp.float32),
        pltpu.SemaphoreType.DMA((2,)),
        pltpu.SemaphoreType.DMA((2,)),
        pltpu.SemaphoreType.DMA((2,)),
        pltpu.SemaphoreType.DMA((4,)),
        pltpu.SemaphoreType.DMA,
    ],
    compiler_params=pltpu.CompilerParams(needs_layout_passes=False),
)(_edge_body)


def _emb_mm_body(an_ref, emb_ref, we_ref, ws_ref, h_ref, hw_ref, hs_ref):
    an = an_ref[...]
    onehot = (an == lax.broadcasted_iota(jnp.int32, (an.shape[0], 128), 1))
    h = jnp.dot(onehot.astype(jnp.float32), emb_ref[...], precision=_HIGH)
    h_ref[...] = h
    hw_ref[...] = jnp.dot(h, we_ref[...])
    hs_ref[...] = jnp.dot(h, ws_ref[...])


def _emb_mm(an2, embp, we, ws):
    return pl.pallas_call(
        _emb_mm_body,
        out_shape=(jax.ShapeDtypeStruct((N, D), jnp.float32),
                   jax.ShapeDtypeStruct((N, D), jnp.float32),
                   jax.ShapeDtypeStruct((N, D), jnp.float32)),
    )(an2, embp, we, ws)


def _rad_body(r2_ref, w1_ref, b1_ref, w2_ref, b2_ref, out_ref):
    r = jnp.sqrt(r2_ref[...] + 1e-8)
    a = jnp.maximum(r * w1_ref[...] + b1_ref[...], 0.0)
    out_ref[...] = jnp.dot(a, w2_ref[...]) + b2_ref[...]


def _rad_layer(r2c, w1, b1, w2, b2):
    BE = 8000
    return pl.pallas_call(
        _rad_body,
        grid=(E // BE,),
        in_specs=[
            pl.BlockSpec((BE, 1), lambda i: (i, 0)),
            pl.BlockSpec((1, RH), lambda i: (0, 0)),
            pl.BlockSpec((1, RH), lambda i: (0, 0)),
            pl.BlockSpec((RH, D), lambda i: (0, 0)),
            pl.BlockSpec((1, D), lambda i: (0, 0)),
        ],
        out_specs=pl.BlockSpec((BE, D), lambda i: (i, 0)),
        out_shape=jax.ShapeDtypeStruct((E, D), jnp.float32),
    )(r2c, w1, b1, w2, b2)


def _mm_body(h_ref, we_ref, ws_ref, hw_ref, hs_ref):
    h = h_ref[...]
    hw_ref[...] = jnp.dot(h, we_ref[...])
    hs_ref[...] = jnp.dot(h, ws_ref[...])


def _node_mm(h, we, ws):
    return pl.pallas_call(
        _mm_body,
        out_shape=(jax.ShapeDtypeStruct((N, D), jnp.float32),
                   jax.ShapeDtypeStruct((N, D), jnp.float32)),
    )(h, we, ws)


def _comb_head_body(agg_ref, hs_ref, w1_ref, b1_ref, w2t_ref, b2_ref,
                    out_ref):
    h = agg_ref[0, :N] + agg_ref[1, :N] + hs_ref[...]
    pooled = jnp.max(h, axis=0, keepdims=True)
    pooled8 = jnp.broadcast_to(pooled, (8, D))
    z = jnp.maximum(jnp.dot(pooled8, w1_ref[...]) + b1_ref[...], 0.0)
    val = jnp.sum(z[0:1, :] * w2t_ref[...], axis=-1, keepdims=True)
    out_ref[...] = val + b2_ref[...]


def _comb_head(agg2, hself, W1, b1r, W2t, b2r):
    return pl.pallas_call(
        _comb_head_body,
        out_shape=jax.ShapeDtypeStruct((1, 1), jnp.float32),
    )(agg2, hself, W1, b1r, W2t, b2r)


def _comb_mm_body(agg_ref, hs_ref, ns_ref, we_ref, ws_ref,
                  h_ref, hw_ref, hs2_ref):
    h = agg_ref[0, :N] + agg_ref[1, :N] + hs_ref[...]
    nrm = jnp.sqrt(jnp.sum(h * h, axis=-1, keepdims=True) + 1e-12)
    h = (h / nrm) * jnp.maximum(ns_ref[...] * nrm, 0.0)
    h_ref[...] = h
    hw_ref[...] = jnp.dot(h, we_ref[...])
    hs2_ref[...] = jnp.dot(h, ws_ref[...])


def _comb_mm(agg2, hself, ns_row, we, ws):
    return pl.pallas_call(
        _comb_mm_body,
        out_shape=(jax.ShapeDtypeStruct((N, D), jnp.float32),
                   jax.ShapeDtypeStruct((N, D), jnp.float32),
                   jax.ShapeDtypeStruct((N, D), jnp.float32)),
    )(agg2, hself, ns_row, we, ws)


@jax.jit
def kernel(pos, edge_index, atomic_numbers, emb, Wedge, Wself, Rw1, Rb1,
           Rw2, Rb2, norm_scale, W1, b1, W2, b2):
    src = edge_index[0].astype(jnp.int32)
    dst = edge_index[1].astype(jnp.int32)
    src3 = src.reshape(NW, NCHUNK, CHUNK)
    dst3 = dst.reshape(NW, NCHUNK, CHUNK)
    an2 = atomic_numbers.astype(jnp.int32).reshape(N, 1)
    embp = jnp.zeros((128, D), jnp.float32).at[:100].set(emb)

    h, hw, hself = _emb_mm(an2, embp, Wedge[0], Wself[0])
    posx = jnp.asarray(pos[:, 0])
    posy = jnp.asarray(pos[:, 1])
    posz = jnp.asarray(pos[:, 2])
    r2 = _r2_call(src, dst, posx, posy, posz)
    r2c = r2.reshape(E, 1)

    rad4s = [
        _rad_layer(r2c, Rw1[l], Rb1[l].reshape(1, RH), Rw2[l],
                   Rb2[l].reshape(1, D)).reshape(NW, NCHUNK, CHUNK, D)
        for l in range(L)
    ]

    for l in range(L):
        agg2 = _edge_call(hw, rad4s[l], src3, dst3)
        if l < L - 1:
            h, hw, hself = _comb_mm(agg2, hself,
                                    norm_scale[l].reshape(1, D),
                                    Wedge[l + 1], Wself[l + 1])
        else:
            out = _comb_head(agg2, hself, W1, b1.reshape(1, D),
                             W2.reshape(1, D), b2.reshape(1, 1))
    return out.reshape(1)
